# baseline (speedup 1.0000x reference)
import functools

import jax
import jax.numpy as jnp
from jax import lax
from jax.experimental import pallas as pl
from jax.experimental.pallas import tpu as pltpu
from jax.experimental.pallas import tpu_sc as plsc

N = 16384
D = 192
H = 384
P = 64
N_SQ = 65
N_KIND = 49
N_PROMO = 5
N_META = 129
T_ROWS = (P + 1) + N_SQ + N_KIND + N_PROMO + N_META

NC = 2
NS = 16
NW = NC * NS
RPW = N // NW
C = 128
NCHUNK = RPW // C
NVEC = D // 16


def _gelu(x):
    return 0.5 * x * (1.0 + lax.erf(x * 0.7071067811865476))


def _sc_gather_sum(table, idx):
    mesh = plsc.VectorSubcoreMesh(core_axis_name="c", subcore_axis_name="s")

    @functools.partial(
        pl.kernel,
        mesh=mesh,
        out_type=jax.ShapeDtypeStruct((N, D), jnp.float32),
        scratch_types=[
            pltpu.VMEM((7, RPW), jnp.int32),
            pltpu.VMEM((RPW, D), jnp.float32),
            pltpu.SemaphoreType.DMA,
        ],
        compiler_params=pltpu.CompilerParams(use_tc_tiling_on_sc=False),
    )
    def k(table_hbm, idx_hbm, out_hbm, idx_v, acc_v, sem):
        wid = lax.axis_index("s") * NC + lax.axis_index("c")
        base = wid * RPW
        pltpu.sync_copy(idx_hbm.at[wid], idx_v)

        fills = [
            pltpu.async_copy(
                table_hbm.at[idx_v.at[0, pl.ds(ci * C, C)]],
                acc_v.at[pl.ds(ci * C, C)], sem)
            for ci in range(NCHUNK)
        ]
        for cp in fills:
            cp.wait()
        adds = [
            pltpu.async_copy(
                table_hbm.at[idx_v.at[t, pl.ds(ci * C, C)]],
                acc_v.at[pl.ds(ci * C, C)], sem, add=True)
            for t in range(1, 7)
            for ci in range(NCHUNK)
        ]
        for cp in adds:
            cp.wait()
        pltpu.sync_copy(acc_v, out_hbm.at[pl.ds(base, RPW)])

    return k(table, idx)


def _tc_body(tok_ref, flags_ref, cons_ref, gctx_ref, fW1_ref, fb1_ref,
             fW2_ref, fb2_ref, cW1_ref, cb1_ref, cW2_ref, cb2_ref,
             ln_g_ref, ln_b_ref, oW1_ref, ob1_ref, oW2_ref, ob2_ref,
             out_ref):
    f32 = jnp.float32
    tok = tok_ref[...] + gctx_ref[...]
    h1 = _gelu(jnp.dot(flags_ref[...], fW1_ref[...],
                       preferred_element_type=f32) + fb1_ref[...])
    tok = tok + jnp.dot(h1, fW2_ref[...], preferred_element_type=f32) + fb2_ref[...]
    h2 = _gelu(jnp.dot(cons_ref[...], cW1_ref[...],
                       preferred_element_type=f32) + cb1_ref[...])
    tok = tok + jnp.dot(h2, cW2_ref[...], preferred_element_type=f32) + cb2_ref[...]
    mu = jnp.mean(tok, axis=-1, keepdims=True)
    cen = tok - mu
    var = jnp.mean(cen * cen, axis=-1, keepdims=True)
    h = cen * jax.lax.rsqrt(var + 1e-5) * ln_g_ref[...] + ln_b_ref[...]
    h3 = _gelu(jnp.dot(h, oW1_ref[...], preferred_element_type=f32) + ob1_ref[...])
    out_ref[...] = (jnp.dot(h3, oW2_ref[...], preferred_element_type=f32)
                    + ob2_ref[...])


def _tc_encode(tok, flags, cons, gctx, fW1, fb1, fW2, fb2, cW1, cb1, cW2,
               cb2, ln_g, ln_b, oW1, ob1, oW2, ob2, block_n=2048):
    grid = (N // block_n,)

    def rows(bn):
        return pl.BlockSpec((bn, None), lambda i: (i, 0))

    def full(shape):
        return pl.BlockSpec(shape, lambda i: tuple(0 for _ in shape))

    rows_spec = pl.BlockSpec((block_n, D), lambda i: (i, 0))
    in_specs = [
        rows_spec,
        pl.BlockSpec((block_n, 7), lambda i: (i, 0)),
        pl.BlockSpec((block_n, 12), lambda i: (i, 0)),
        full((1, D)),
        full((7, H)), full((1, H)), full((H, D)), full((1, D)),
        full((12, H)), full((1, H)), full((H, D)), full((1, D)),
        full((1, D)), full((1, D)),
        full((D, H)), full((1, H)), full((H, D)), full((1, D)),
    ]
    return pl.pallas_call(
        _tc_body,
        grid=grid,
        in_specs=in_specs,
        out_specs=rows_spec,
        out_shape=jax.ShapeDtypeStruct((N, D), jnp.float32),
    )(tok, flags, cons, gctx, fW1, fb1, fW2, fb2, cW1, cb1, cW2, cb2,
      ln_g, ln_b, oW1, ob1, oW2, ob2)


def kernel(global_context, piece_context, flags, consequence, square_emb,
           kind_emb, promo_emb, meta_emb, null_piece, fW1, fb1, fW2, fb2,
           cW1, cb1, cW2, cb2, ln_g, ln_b, oW1, ob1, oW2, ob2, moving_idx,
           target_idx, from_sq, to_sq, move_kind, promo_idx, meta_idx):
    piece_full = jnp.concatenate([piece_context, null_piece[None, :]], axis=0)
    table = jnp.concatenate(
        [piece_full, square_emb, kind_emb, promo_emb, meta_emb], axis=0)
    o_sq = P + 1
    o_kind = o_sq + N_SQ
    o_promo = o_kind + N_KIND
    o_meta = o_promo + N_PROMO
    i32 = jnp.int32
    idx = jnp.stack([
        moving_idx.astype(i32),
        target_idx.astype(i32),
        from_sq.astype(i32) + o_sq,
        to_sq.astype(i32) + o_sq,
        move_kind.astype(i32) + o_kind,
        promo_idx.astype(i32) + o_promo,
        meta_idx.astype(i32) + o_meta,
    ])
    idx = idx.reshape(7, NW, RPW).transpose(1, 0, 2)

    tok = _sc_gather_sum(table, idx)

    r1 = lambda v: v[None, :]
    return _tc_encode(tok, flags, consequence, r1(global_context),
                      fW1, r1(fb1), fW2, r1(fb2),
                      cW1, r1(cb1), cW2, r1(cb2),
                      r1(ln_g), r1(ln_b),
                      oW1, r1(ob1), oW2, r1(ob2))


# speedup vs baseline: 1.0624x; 1.0624x over previous
import functools

import jax
import jax.numpy as jnp
from jax import lax
from jax.experimental import pallas as pl
from jax.experimental.pallas import tpu as pltpu
from jax.experimental.pallas import tpu_sc as plsc

N = 16384
D = 192
H = 384
P = 64
N_SQ = 65
N_KIND = 49
N_PROMO = 5
N_META = 129

NC = 2
NS = 16
NW = NC * NS
RPW = N // NW
C = 128
NCHUNK = RPW // C
NSTR = 3


def _gelu(x):
    return 0.5 * x * (1.0 + lax.erf(x * 0.7071067811865476))


def _sc_gather_sum(table, idx):
    mesh = plsc.VectorSubcoreMesh(core_axis_name="c", subcore_axis_name="s")

    @functools.partial(
        pl.kernel,
        mesh=mesh,
        out_type=jax.ShapeDtypeStruct((N, D), jnp.float32),
        scratch_types=[
            pltpu.VMEM((NSTR, RPW), jnp.int32),
            pltpu.VMEM((RPW, D), jnp.float32),
            pltpu.SemaphoreType.DMA,
        ],
        compiler_params=pltpu.CompilerParams(use_tc_tiling_on_sc=False),
    )
    def k(table_hbm, idx_hbm, out_hbm, idx_v, acc_v, sem):
        wid = lax.axis_index("s") * NC + lax.axis_index("c")
        base = wid * RPW
        pltpu.sync_copy(idx_hbm.at[wid], idx_v)

        fills = [
            pltpu.async_copy(
                table_hbm.at[idx_v.at[0, pl.ds(ci * C, C)]],
                acc_v.at[pl.ds(ci * C, C)], sem)
            for ci in range(NCHUNK)
        ]
        for cp in fills:
            cp.wait()
        adds = [
            pltpu.async_copy(
                table_hbm.at[idx_v.at[t, pl.ds(ci * C, C)]],
                acc_v.at[pl.ds(ci * C, C)], sem, add=True)
            for t in range(1, NSTR)
            for ci in range(NCHUNK)
        ]
        for cp in adds:
            cp.wait()
        pltpu.sync_copy(acc_v, out_hbm.at[pl.ds(base, RPW)])

    return k(table, idx)


def _tc_body(tok_ref, flags_ref, cons_ref, fW1_ref, fb1_ref,
             fW2_ref, fb2_ref, cW1_ref, cb1_ref, cW2_ref, cb2_ref,
             ln_g_ref, ln_b_ref, oW1_ref, ob1_ref, oW2_ref, ob2_ref,
             out_ref):
    f32 = jnp.float32
    tok = tok_ref[...]
    h1 = _gelu(jnp.dot(flags_ref[...], fW1_ref[...],
                       preferred_element_type=f32) + fb1_ref[...])
    tok = tok + jnp.dot(h1, fW2_ref[...], preferred_element_type=f32) + fb2_ref[...]
    h2 = _gelu(jnp.dot(cons_ref[...], cW1_ref[...],
                       preferred_element_type=f32) + cb1_ref[...])
    tok = tok + jnp.dot(h2, cW2_ref[...], preferred_element_type=f32) + cb2_ref[...]
    mu = jnp.mean(tok, axis=-1, keepdims=True)
    cen = tok - mu
    var = jnp.mean(cen * cen, axis=-1, keepdims=True)
    h = cen * jax.lax.rsqrt(var + 1e-5) * ln_g_ref[...] + ln_b_ref[...]
    h3 = _gelu(jnp.dot(h, oW1_ref[...], preferred_element_type=f32) + ob1_ref[...])
    out_ref[...] = (jnp.dot(h3, oW2_ref[...], preferred_element_type=f32)
                    + ob2_ref[...])


def _tc_encode(tok, flags, cons, fW1, fb1, fW2, fb2, cW1, cb1, cW2,
               cb2, ln_g, ln_b, oW1, ob1, oW2, ob2, block_n=2048):
    grid = (N // block_n,)

    def full(shape):
        return pl.BlockSpec(shape, lambda i: tuple(0 for _ in shape))

    rows_spec = pl.BlockSpec((block_n, D), lambda i: (i, 0))
    in_specs = [
        rows_spec,
        pl.BlockSpec((block_n, 7), lambda i: (i, 0)),
        pl.BlockSpec((block_n, 12), lambda i: (i, 0)),
        full((7, H)), full((1, H)), full((H, D)), full((1, D)),
        full((12, H)), full((1, H)), full((H, D)), full((1, D)),
        full((1, D)), full((1, D)),
        full((D, H)), full((1, H)), full((H, D)), full((1, D)),
    ]
    return pl.pallas_call(
        _tc_body,
        grid=grid,
        in_specs=in_specs,
        out_specs=rows_spec,
        out_shape=jax.ShapeDtypeStruct((N, D), jnp.float32),
    )(tok, flags, cons, fW1, fb1, fW2, fb2, cW1, cb1, cW2, cb2,
      ln_g, ln_b, oW1, ob1, oW2, ob2)


def kernel(global_context, piece_context, flags, consequence, square_emb,
           kind_emb, promo_emb, meta_emb, null_piece, fW1, fb1, fW2, fb2,
           cW1, cb1, cW2, cb2, ln_g, ln_b, oW1, ob1, oW2, ob2, moving_idx,
           target_idx, from_sq, to_sq, move_kind, promo_idx, meta_idx):
    piece_full = jnp.concatenate([piece_context, null_piece[None, :]], axis=0)
    pp = (piece_context[:, None, :] + piece_full[None, :, :]).reshape(-1, D)
    ss = (square_emb[:, None, :] + square_emb[None, :, :]).reshape(-1, D)
    kpm = (kind_emb[:, None, None, :] + promo_emb[None, :, None, :]
           + (meta_emb + global_context)[None, None, :, :]).reshape(-1, D)
    table = jnp.concatenate([pp, ss, kpm], axis=0)
    o_ss = pp.shape[0]
    o_kpm = o_ss + ss.shape[0]
    i32 = jnp.int32
    idx = jnp.stack([
        moving_idx.astype(i32) * (P + 1) + target_idx.astype(i32),
        from_sq.astype(i32) * N_SQ + to_sq.astype(i32) + o_ss,
        (move_kind.astype(i32) * N_PROMO + promo_idx.astype(i32)) * N_META
        + meta_idx.astype(i32) + o_kpm,
    ])
    idx = idx.reshape(NSTR, NW, RPW).transpose(1, 0, 2)

    tok = _sc_gather_sum(table, idx)

    r1 = lambda v: v[None, :]
    return _tc_encode(tok, flags, consequence,
                      fW1, r1(fb1), fW2, r1(fb2),
                      cW1, r1(cb1), cW2, r1(cb2),
                      r1(ln_g), r1(ln_b),
                      oW1, r1(ob1), oW2, r1(ob2))


# speedup vs baseline: 2.0415x; 1.9217x over previous
import functools

import jax
import jax.numpy as jnp
from jax import lax
from jax.experimental import pallas as pl
from jax.experimental.pallas import tpu as pltpu
from jax.experimental.pallas import tpu_sc as plsc

N = 16384
D = 192
H = 384
P = 64
N_SQ = 65
N_KIND = 49
N_PROMO = 5
N_META = 129

NC = 2
NS = 16
NW = NC * NS
RPW = N // NW
C = 128
NCHUNK = RPW // C
NSTR = 4


def _gelu(x):
    return 0.5 * x * (1.0 + lax.erf(x * 0.7071067811865476))


def _sc_gather_sum(tables, idx):
    mesh = plsc.VectorSubcoreMesh(core_axis_name="c", subcore_axis_name="s")

    @functools.partial(
        pl.kernel,
        mesh=mesh,
        out_type=jax.ShapeDtypeStruct((N, D), jnp.float32),
        scratch_types=[
            pltpu.VMEM((NSTR, RPW), jnp.int32),
            pltpu.VMEM((RPW, D), jnp.float32),
            pltpu.SemaphoreType.DMA,
        ],
        compiler_params=pltpu.CompilerParams(use_tc_tiling_on_sc=False),
    )
    def k(t0_hbm, t1_hbm, t2_hbm, t3_hbm, idx_hbm, out_hbm, idx_v, acc_v,
          sem):
        tabs = (t0_hbm, t1_hbm, t2_hbm, t3_hbm)
        wid = lax.axis_index("s") * NC + lax.axis_index("c")
        base = wid * RPW
        pltpu.sync_copy(idx_hbm.at[wid], idx_v)

        fills = [
            pltpu.async_copy(
                tabs[0].at[idx_v.at[0, pl.ds(ci * C, C)]],
                acc_v.at[pl.ds(ci * C, C)], sem)
            for ci in range(NCHUNK)
        ]
        for cp in fills:
            cp.wait()
        adds = [
            pltpu.async_copy(
                tabs[t].at[idx_v.at[t, pl.ds(ci * C, C)]],
                acc_v.at[pl.ds(ci * C, C)], sem, add=True)
            for t in range(1, NSTR)
            for ci in range(NCHUNK)
        ]
        for cp in adds:
            cp.wait()
        pltpu.sync_copy(acc_v, out_hbm.at[pl.ds(base, RPW)])

    return k(*tables, idx)


def _tc_body(tok_ref, flags_ref, cons_ref, fW1_ref, fb1_ref,
             fW2_ref, fb2_ref, cW1_ref, cb1_ref, cW2_ref, cb2_ref,
             ln_g_ref, ln_b_ref, oW1_ref, ob1_ref, oW2_ref, ob2_ref,
             out_ref):
    f32 = jnp.float32
    tok = tok_ref[...]
    h1 = _gelu(jnp.dot(flags_ref[...], fW1_ref[...],
                       preferred_element_type=f32) + fb1_ref[...])
    tok = tok + jnp.dot(h1, fW2_ref[...], preferred_element_type=f32) + fb2_ref[...]
    h2 = _gelu(jnp.dot(cons_ref[...], cW1_ref[...],
                       preferred_element_type=f32) + cb1_ref[...])
    tok = tok + jnp.dot(h2, cW2_ref[...], preferred_element_type=f32) + cb2_ref[...]
    mu = jnp.mean(tok, axis=-1, keepdims=True)
    cen = tok - mu
    var = jnp.mean(cen * cen, axis=-1, keepdims=True)
    h = cen * jax.lax.rsqrt(var + 1e-5) * ln_g_ref[...] + ln_b_ref[...]
    h3 = _gelu(jnp.dot(h, oW1_ref[...], preferred_element_type=f32) + ob1_ref[...])
    out_ref[...] = (jnp.dot(h3, oW2_ref[...], preferred_element_type=f32)
                    + ob2_ref[...])


def _tc_encode(tok, flags, cons, fW1, fb1, fW2, fb2, cW1, cb1, cW2,
               cb2, ln_g, ln_b, oW1, ob1, oW2, ob2, block_n=2048):
    grid = (N // block_n,)

    def full(shape):
        return pl.BlockSpec(shape, lambda i: tuple(0 for _ in shape))

    rows_spec = pl.BlockSpec((block_n, D), lambda i: (i, 0))
    in_specs = [
        rows_spec,
        pl.BlockSpec((block_n, 7), lambda i: (i, 0)),
        pl.BlockSpec((block_n, 12), lambda i: (i, 0)),
        full((7, H)), full((1, H)), full((H, D)), full((1, D)),
        full((12, H)), full((1, H)), full((H, D)), full((1, D)),
        full((1, D)), full((1, D)),
        full((D, H)), full((1, H)), full((H, D)), full((1, D)),
    ]
    return pl.pallas_call(
        _tc_body,
        grid=grid,
        in_specs=in_specs,
        out_specs=rows_spec,
        out_shape=jax.ShapeDtypeStruct((N, D), jnp.float32),
    )(tok, flags, cons, fW1, fb1, fW2, fb2, cW1, cb1, cW2, cb2,
      ln_g, ln_b, oW1, ob1, oW2, ob2)


def kernel(global_context, piece_context, flags, consequence, square_emb,
           kind_emb, promo_emb, meta_emb, null_piece, fW1, fb1, fW2, fb2,
           cW1, cb1, cW2, cb2, ln_g, ln_b, oW1, ob1, oW2, ob2, moving_idx,
           target_idx, from_sq, to_sq, move_kind, promo_idx, meta_idx):
    piece_full = jnp.concatenate([piece_context, null_piece[None, :]], axis=0)
    pp = (piece_context[:, None, :] + piece_full[None, :, :]).reshape(-1, D)
    ss = (square_emb[:, None, :] + square_emb[None, :, :]).reshape(-1, D)
    kp = (kind_emb[:, None, :] + promo_emb[None, :, :]).reshape(-1, D)
    me = meta_emb + global_context
    i32 = jnp.int32
    idx = jnp.stack([
        moving_idx.astype(i32) * (P + 1) + target_idx.astype(i32),
        from_sq.astype(i32) * N_SQ + to_sq.astype(i32),
        move_kind.astype(i32) * N_PROMO + promo_idx.astype(i32),
        meta_idx.astype(i32),
    ])
    idx = idx.reshape(NSTR, NW, RPW).transpose(1, 0, 2)

    tok = _sc_gather_sum((pp, ss, kp, me), idx)

    r1 = lambda v: v[None, :]
    return _tc_encode(tok, flags, consequence,
                      fW1, r1(fb1), fW2, r1(fb2),
                      cW1, r1(cb1), cW2, r1(cb2),
                      r1(ln_g), r1(ln_b),
                      oW1, r1(ob1), oW2, r1(ob2))


# speedup vs baseline: 2.4140x; 1.1825x over previous
import functools

import jax
import jax.numpy as jnp
from jax import lax
from jax.experimental import pallas as pl
from jax.experimental.pallas import tpu as pltpu
from jax.experimental.pallas import tpu_sc as plsc

N = 16384
D = 192
H = 384
P = 64
N_SQ = 65
N_KIND = 49
N_PROMO = 5
N_META = 129

NC = 2
NS = 16
NW = NC * NS
RPW = N // NW
C = 128
NCHUNK = RPW // C
NSTR = 4
VL = 16


def _gelu(x):
    return 0.5 * x * (1.0 + lax.erf(x * 0.7071067811865476))


def _sc_gather_sum(tables, raw_idx):
    mesh = plsc.VectorSubcoreMesh(core_axis_name="c", subcore_axis_name="s")

    @functools.partial(
        pl.kernel,
        mesh=mesh,
        out_type=(jax.ShapeDtypeStruct((N, 128), jnp.float32),
                  jax.ShapeDtypeStruct((N, 128), jnp.float32)),
        scratch_types=[
            pltpu.VMEM((6, RPW), jnp.int32),
            pltpu.VMEM((NSTR, RPW), jnp.int32),
            pltpu.VMEM((RPW, D), jnp.float32),
            pltpu.SemaphoreType.DMA,
        ],
        compiler_params=pltpu.CompilerParams(use_tc_tiling_on_sc=False),
    )
    def k(t0_hbm, t1_hbm, t2_hbm, t3_hbm, mov_hbm, tgt_hbm, frm_hbm,
          to_hbm, knd_hbm, prm_hbm, met_hbm, outa_hbm, outb_hbm, raw_v,
          idx_v, acc_v, sem):
        tabs = (t0_hbm, t1_hbm, t2_hbm, t3_hbm)
        wid = lax.axis_index("s") * NC + lax.axis_index("c")
        base = wid * RPW
        sl = pl.ds(base, RPW)
        incs = [
            pltpu.async_copy(r.at[sl], raw_v.at[j], sem)
            for j, r in enumerate(
                (mov_hbm, tgt_hbm, frm_hbm, to_hbm, knd_hbm, prm_hbm))
        ]
        incs.append(pltpu.async_copy(met_hbm.at[sl], idx_v.at[3], sem))
        for cp in incs:
            cp.wait()
        for i in range(RPW // VL):
            v = pl.ds(i * VL, VL)
            idx_v[0, v] = raw_v[0, v] * (P + 1) + raw_v[1, v]
            idx_v[1, v] = raw_v[2, v] * N_SQ + raw_v[3, v]
            idx_v[2, v] = raw_v[4, v] * N_PROMO + raw_v[5, v]

        fills = [
            pltpu.async_copy(
                tabs[0].at[idx_v.at[0, pl.ds(ci * C, C)]],
                acc_v.at[pl.ds(ci * C, C)], sem)
            for ci in range(NCHUNK)
        ]
        for cp in fills:
            cp.wait()
        adds = [
            pltpu.async_copy(
                tabs[t].at[idx_v.at[t, pl.ds(ci * C, C)]],
                acc_v.at[pl.ds(ci * C, C)], sem, add=True)
            for t in range(1, NSTR)
            for ci in range(NCHUNK)
        ]
        for cp in adds:
            cp.wait()
        pltpu.sync_copy(acc_v.at[:, pl.ds(0, 128)], outa_hbm.at[sl])
        pltpu.sync_copy(acc_v.at[:, pl.ds(128, 64)],
                        outb_hbm.at[sl, pl.ds(0, 64)])

    return k(*tables, *raw_idx)


def _tc_body(toka_ref, tokb_ref, flags_ref, cons_ref, fW1_ref, fb1_ref,
             fW2_ref, fb2_ref, cW1_ref, cb1_ref, cW2_ref, cb2_ref,
             ln_g_ref, ln_b_ref, oW1_ref, ob1_ref, oW2_ref, ob2_ref,
             out_ref):
    f32 = jnp.float32
    tok = jnp.concatenate([toka_ref[...], tokb_ref[:, :64]], axis=1)
    h1 = _gelu(jnp.dot(flags_ref[...], fW1_ref[...],
                       preferred_element_type=f32) + fb1_ref[...])
    tok = tok + jnp.dot(h1, fW2_ref[...], preferred_element_type=f32) + fb2_ref[...]
    h2 = _gelu(jnp.dot(cons_ref[...], cW1_ref[...],
                       preferred_element_type=f32) + cb1_ref[...])
    tok = tok + jnp.dot(h2, cW2_ref[...], preferred_element_type=f32) + cb2_ref[...]
    mu = jnp.mean(tok, axis=-1, keepdims=True)
    cen = tok - mu
    var = jnp.mean(cen * cen, axis=-1, keepdims=True)
    h = cen * jax.lax.rsqrt(var + 1e-5) * ln_g_ref[...] + ln_b_ref[...]
    h3 = _gelu(jnp.dot(h, oW1_ref[...], preferred_element_type=f32) + ob1_ref[...])
    out_ref[...] = (jnp.dot(h3, oW2_ref[...], preferred_element_type=f32)
                    + ob2_ref[...])


def _tc_encode(toka, tokb, flags, cons, fW1, fb1, fW2, fb2, cW1, cb1, cW2,
               cb2, ln_g, ln_b, oW1, ob1, oW2, ob2, block_n=2048):
    grid = (N // block_n,)

    def full(shape):
        return pl.BlockSpec(shape, lambda i: tuple(0 for _ in shape))

    rows_spec = pl.BlockSpec((block_n, D), lambda i: (i, 0))
    half_spec = pl.BlockSpec((block_n, 128), lambda i: (i, 0))
    in_specs = [
        half_spec, half_spec,
        pl.BlockSpec((block_n, 7), lambda i: (i, 0)),
        pl.BlockSpec((block_n, 12), lambda i: (i, 0)),
        full((7, H)), full((1, H)), full((H, D)), full((1, D)),
        full((12, H)), full((1, H)), full((H, D)), full((1, D)),
        full((1, D)), full((1, D)),
        full((D, H)), full((1, H)), full((H, D)), full((1, D)),
    ]
    return pl.pallas_call(
        _tc_body,
        grid=grid,
        in_specs=in_specs,
        out_specs=rows_spec,
        out_shape=jax.ShapeDtypeStruct((N, D), jnp.float32),
    )(toka, tokb, flags, cons, fW1, fb1, fW2, fb2, cW1, cb1, cW2, cb2,
      ln_g, ln_b, oW1, ob1, oW2, ob2)


def kernel(global_context, piece_context, flags, consequence, square_emb,
           kind_emb, promo_emb, meta_emb, null_piece, fW1, fb1, fW2, fb2,
           cW1, cb1, cW2, cb2, ln_g, ln_b, oW1, ob1, oW2, ob2, moving_idx,
           target_idx, from_sq, to_sq, move_kind, promo_idx, meta_idx):
    piece_full = jnp.concatenate([piece_context, null_piece[None, :]], axis=0)
    pp = (piece_context[:, None, :] + piece_full[None, :, :]).reshape(-1, D)
    ss = (square_emb[:, None, :] + square_emb[None, :, :]).reshape(-1, D)
    kp = (kind_emb[:, None, :] + promo_emb[None, :, :]).reshape(-1, D)
    me = meta_emb + global_context
    i32 = jnp.int32
    raw = (moving_idx.astype(i32), target_idx.astype(i32),
           from_sq.astype(i32), to_sq.astype(i32),
           move_kind.astype(i32), promo_idx.astype(i32),
           meta_idx.astype(i32))

    toka, tokb = _sc_gather_sum((pp, ss, kp, me), raw)

    r1 = lambda v: v[None, :]
    return _tc_encode(toka, tokb, flags, consequence,
                      fW1, r1(fb1), fW2, r1(fb2),
                      cW1, r1(cb1), cW2, r1(cb2),
                      r1(ln_g), r1(ln_b),
                      oW1, r1(ob1), oW2, r1(ob2))


# speedup vs baseline: 2.9692x; 1.2300x over previous
import functools

import jax
import jax.numpy as jnp
from jax import lax
from jax.experimental import pallas as pl
from jax.experimental.pallas import tpu as pltpu
from jax.experimental.pallas import tpu_sc as plsc

N = 16384
D = 192
H = 384
P = 64
N_SQ = 65
N_KIND = 49
N_PROMO = 5
N_META = 129

NC = 2
NS = 16
NW = NC * NS
RPW = N // NW
C = 128
NCHUNK = RPW // C
NSTR = 4
VL = 16


def _gelu(x):
    return 0.5 * x * (1.0 + lax.erf(x * 0.7071067811865476))


def _sc_gather_sum(tables, raw_idx):
    mesh = plsc.VectorSubcoreMesh(core_axis_name="c", subcore_axis_name="s")

    @functools.partial(
        pl.kernel,
        mesh=mesh,
        out_type=(jax.ShapeDtypeStruct((N, 128), jnp.float32),
                  jax.ShapeDtypeStruct((N, 128), jnp.float32)),
        scratch_types=[
            pltpu.VMEM((6, RPW), jnp.int32),
            pltpu.VMEM((NSTR, RPW), jnp.int32),
            pltpu.VMEM((RPW, D), jnp.float32),
            pltpu.SemaphoreType.DMA,
        ],
        compiler_params=pltpu.CompilerParams(use_tc_tiling_on_sc=False),
    )
    def k(t0_hbm, t1_hbm, t2_hbm, t3_hbm, mov_hbm, tgt_hbm, frm_hbm,
          to_hbm, knd_hbm, prm_hbm, met_hbm, outa_hbm, outb_hbm, raw_v,
          idx_v, acc_v, sem):
        tabs = (t0_hbm, t1_hbm, t2_hbm, t3_hbm)
        wid = lax.axis_index("s") * NC + lax.axis_index("c")
        base = wid * RPW
        sl = pl.ds(base, RPW)
        incs = [
            pltpu.async_copy(r.at[sl], raw_v.at[j], sem)
            for j, r in enumerate(
                (mov_hbm, tgt_hbm, frm_hbm, to_hbm, knd_hbm, prm_hbm))
        ]
        incs.append(pltpu.async_copy(met_hbm.at[sl], idx_v.at[3], sem))
        for cp in incs:
            cp.wait()
        for i in range(RPW // VL):
            v = pl.ds(i * VL, VL)
            idx_v[0, v] = raw_v[0, v] * (P + 1) + raw_v[1, v]
            idx_v[1, v] = raw_v[2, v] * N_SQ + raw_v[3, v]
            idx_v[2, v] = raw_v[4, v] * N_PROMO + raw_v[5, v]

        fills = [
            pltpu.async_copy(
                tabs[0].at[idx_v.at[0, pl.ds(ci * C, C)]],
                acc_v.at[pl.ds(ci * C, C)], sem)
            for ci in range(NCHUNK)
        ]
        for cp in fills:
            cp.wait()
        adds = [
            pltpu.async_copy(
                tabs[t].at[idx_v.at[t, pl.ds(ci * C, C)]],
                acc_v.at[pl.ds(ci * C, C)], sem, add=True)
            for t in range(1, NSTR)
            for ci in range(NCHUNK)
        ]
        for cp in adds:
            cp.wait()
        pltpu.sync_copy(acc_v.at[:, pl.ds(0, 128)], outa_hbm.at[sl])
        pltpu.sync_copy(acc_v.at[:, pl.ds(128, 64)],
                        outb_hbm.at[sl, pl.ds(0, 64)])

    return k(*tables, *raw_idx)


def _tc_body(toka_ref, tokb_ref, flags_ref, cons_ref, fW1_ref, fb1_ref,
             fW2_ref, fb2_ref, cW1_ref, cb1_ref, cW2_ref, cb2_ref,
             ln_g_ref, ln_b_ref, oW1_ref, ob1_ref, oW2_ref, ob2_ref,
             out_ref):
    f32 = jnp.float32
    t_lhs = (((0,), (0,)), ((), ()))
    tok = jnp.concatenate([toka_ref[...], tokb_ref[:, :64]], axis=1)
    h1 = _gelu(lax.dot_general(flags_ref[...], fW1_ref[...], t_lhs,
                               preferred_element_type=f32) + fb1_ref[...])
    tok = tok + jnp.dot(h1, fW2_ref[...], preferred_element_type=f32) + fb2_ref[...]
    h2 = _gelu(lax.dot_general(cons_ref[...], cW1_ref[...], t_lhs,
                               preferred_element_type=f32) + cb1_ref[...])
    tok = tok + jnp.dot(h2, cW2_ref[...], preferred_element_type=f32) + cb2_ref[...]
    mu = jnp.mean(tok, axis=-1, keepdims=True)
    cen = tok - mu
    var = jnp.mean(cen * cen, axis=-1, keepdims=True)
    h = cen * jax.lax.rsqrt(var + 1e-5) * ln_g_ref[...] + ln_b_ref[...]
    h3 = _gelu(jnp.dot(h, oW1_ref[...], preferred_element_type=f32) + ob1_ref[...])
    out_ref[...] = (lax.dot_general(oW2_ref[...], h3,
                                    (((0,), (1,)), ((), ())),
                                    preferred_element_type=f32)
                    + ob2_ref[...])


def _tc_encode(toka, tokb, flags, cons, fW1, fb1, fW2, fb2, cW1, cb1, cW2,
               cb2, ln_g, ln_b, oW1, ob1, oW2, ob2, block_n=2048):
    grid = (N // block_n,)

    def full(shape):
        return pl.BlockSpec(shape, lambda i: tuple(0 for _ in shape))

    half_spec = pl.BlockSpec((block_n, 128), lambda i: (i, 0))
    in_specs = [
        half_spec, half_spec,
        pl.BlockSpec((7, block_n), lambda i: (0, i)),
        pl.BlockSpec((12, block_n), lambda i: (0, i)),
        full((7, H)), full((1, H)), full((H, D)), full((1, D)),
        full((12, H)), full((1, H)), full((H, D)), full((1, D)),
        full((1, D)), full((1, D)),
        full((D, H)), full((1, H)), full((H, D)), full((D, 1)),
    ]
    return pl.pallas_call(
        _tc_body,
        grid=grid,
        in_specs=in_specs,
        out_specs=pl.BlockSpec((D, block_n), lambda i: (0, i)),
        out_shape=jax.ShapeDtypeStruct((D, N), jnp.float32),
    )(toka, tokb, flags, cons, fW1, fb1, fW2, fb2, cW1, cb1, cW2, cb2,
      ln_g, ln_b, oW1, ob1, oW2, ob2)


def kernel(global_context, piece_context, flags, consequence, square_emb,
           kind_emb, promo_emb, meta_emb, null_piece, fW1, fb1, fW2, fb2,
           cW1, cb1, cW2, cb2, ln_g, ln_b, oW1, ob1, oW2, ob2, moving_idx,
           target_idx, from_sq, to_sq, move_kind, promo_idx, meta_idx):
    piece_full = jnp.concatenate([piece_context, null_piece[None, :]], axis=0)
    pp = (piece_context[:, None, :] + piece_full[None, :, :]).reshape(-1, D)
    ss = (square_emb[:, None, :] + square_emb[None, :, :]).reshape(-1, D)
    kp = (kind_emb[:, None, :] + promo_emb[None, :, :]).reshape(-1, D)
    me = meta_emb + global_context
    i32 = jnp.int32
    raw = (moving_idx.astype(i32), target_idx.astype(i32),
           from_sq.astype(i32), to_sq.astype(i32),
           move_kind.astype(i32), promo_idx.astype(i32),
           meta_idx.astype(i32))

    toka, tokb = _sc_gather_sum((pp, ss, kp, me), raw)

    r1 = lambda v: v[None, :]
    out_t = _tc_encode(toka, tokb, flags.T, consequence.T,
                       fW1, r1(fb1), fW2, r1(fb2),
                       cW1, r1(cb1), cW2, r1(cb2),
                       r1(ln_g), r1(ln_b),
                       oW1, r1(ob1), oW2, ob2[:, None])
    return out_t.T


# speedup vs baseline: 3.0096x; 1.0136x over previous
import functools

import jax
import jax.numpy as jnp
from jax import lax
from jax.experimental import pallas as pl
from jax.experimental.pallas import tpu as pltpu
from jax.experimental.pallas import tpu_sc as plsc

N = 16384
D = 192
H = 384
P = 64
N_SQ = 65
N_KIND = 49
N_PROMO = 5
N_META = 129

NC = 2
NS = 16
NW = NC * NS
RPW = N // NW
C = 128
NCHUNK = RPW // C
NSTR = 4
VL = 16


def _gelu(x):
    return 0.5 * x * (1.0 + lax.erf(x * 0.7071067811865476))


def _sc_gather_sum(tables, raw_idx):
    mesh = plsc.VectorSubcoreMesh(core_axis_name="c", subcore_axis_name="s")

    @functools.partial(
        pl.kernel,
        mesh=mesh,
        out_type=(jax.ShapeDtypeStruct((N, 128), jnp.float32),
                  jax.ShapeDtypeStruct((N, 128), jnp.float32)),
        scratch_types=[
            pltpu.VMEM((6, RPW), jnp.int32),
            pltpu.VMEM((NSTR, RPW), jnp.int32),
            pltpu.VMEM((RPW, D), jnp.float32),
            pltpu.SemaphoreType.DMA,
        ],
        compiler_params=pltpu.CompilerParams(use_tc_tiling_on_sc=False),
    )
    def k(t0_hbm, t1_hbm, t2_hbm, t3_hbm, mov_hbm, tgt_hbm, frm_hbm,
          to_hbm, knd_hbm, prm_hbm, met_hbm, outa_hbm, outb_hbm, raw_v,
          idx_v, acc_v, sem):
        tabs = (t0_hbm, t1_hbm, t2_hbm, t3_hbm)
        wid = lax.axis_index("s") * NC + lax.axis_index("c")
        base = wid * RPW
        sl = pl.ds(base, RPW)
        incs = [
            pltpu.async_copy(r.at[sl], raw_v.at[j], sem)
            for j, r in enumerate(
                (mov_hbm, tgt_hbm, frm_hbm, to_hbm, knd_hbm, prm_hbm))
        ]
        incs.append(pltpu.async_copy(met_hbm.at[sl], idx_v.at[3], sem))
        for cp in incs:
            cp.wait()
        for i in range(RPW // VL):
            v = pl.ds(i * VL, VL)
            idx_v[0, v] = raw_v[0, v] * (P + 1) + raw_v[1, v]
            idx_v[1, v] = raw_v[2, v] * N_SQ + raw_v[3, v]
            idx_v[2, v] = raw_v[4, v] * N_PROMO + raw_v[5, v]

        fills = [
            pltpu.async_copy(
                tabs[0].at[idx_v.at[0, pl.ds(ci * C, C)]],
                acc_v.at[pl.ds(ci * C, C)], sem)
            for ci in range(NCHUNK)
        ]
        for cp in fills:
            cp.wait()
        adds = [
            pltpu.async_copy(
                tabs[t].at[idx_v.at[t, pl.ds(ci * C, C)]],
                acc_v.at[pl.ds(ci * C, C)], sem, add=True)
            for t in range(1, NSTR)
            for ci in range(NCHUNK)
        ]
        for cp in adds:
            cp.wait()
        pltpu.sync_copy(acc_v.at[:, pl.ds(0, 128)], outa_hbm.at[sl])
        pltpu.sync_copy(acc_v.at[:, pl.ds(128, 64)],
                        outb_hbm.at[sl, pl.ds(0, 64)])

    return k(*tables, *raw_idx)


def _tc_body(toka_ref, tokb_ref, flags_ref, cons_ref, fW1_ref, fb1_ref,
             fW2_ref, fb2_ref, cW1_ref, cb1_ref, cW2_ref, cb2_ref,
             ln_g_ref, ln_b_ref, oW1_ref, ob1_ref, oW2_ref, ob2_ref,
             out_ref):
    f32 = jnp.float32
    t_lhs = (((0,), (0,)), ((), ()))
    t_rhs = (((1,), (1,)), ((), ()))
    tok = jnp.concatenate([toka_ref[...], tokb_ref[:, :64]], axis=1)
    h1 = _gelu(lax.dot_general(flags_ref[...], fW1_ref[...], t_lhs,
                               preferred_element_type=f32) + fb1_ref[...])
    tok = tok + lax.dot_general(h1, fW2_ref[...], t_rhs,
                                preferred_element_type=f32) + fb2_ref[...]
    h2 = _gelu(lax.dot_general(cons_ref[...], cW1_ref[...], t_lhs,
                               preferred_element_type=f32) + cb1_ref[...])
    tok = tok + lax.dot_general(h2, cW2_ref[...], t_rhs,
                                preferred_element_type=f32) + cb2_ref[...]
    mu = jnp.mean(tok, axis=-1, keepdims=True)
    cen = tok - mu
    var = jnp.mean(cen * cen, axis=-1, keepdims=True)
    h = cen * jax.lax.rsqrt(var + 1e-5) * ln_g_ref[...] + ln_b_ref[...]
    h3 = _gelu(jnp.dot(h, oW1_ref[...], preferred_element_type=f32) + ob1_ref[...])
    out_ref[...] = (lax.dot_general(oW2_ref[...], h3,
                                    (((1,), (1,)), ((), ())),
                                    preferred_element_type=f32)
                    + ob2_ref[...])


def _tc_encode(toka, tokb, flags, cons, fW1, fb1, fW2, fb2, cW1, cb1, cW2,
               cb2, ln_g, ln_b, oW1, ob1, oW2, ob2, block_n=4096):
    grid = (N // block_n,)

    def full(shape):
        return pl.BlockSpec(shape, lambda i: tuple(0 for _ in shape))

    half_spec = pl.BlockSpec((block_n, 128), lambda i: (i, 0))
    in_specs = [
        half_spec, half_spec,
        pl.BlockSpec((7, block_n), lambda i: (0, i)),
        pl.BlockSpec((12, block_n), lambda i: (0, i)),
        full((7, H)), full((1, H)), full((D, H)), full((1, D)),
        full((12, H)), full((1, H)), full((D, H)), full((1, D)),
        full((1, D)), full((1, D)),
        full((D, H)), full((1, H)), full((D, H)), full((D, 1)),
    ]
    return pl.pallas_call(
        _tc_body,
        grid=grid,
        in_specs=in_specs,
        out_specs=pl.BlockSpec((D, block_n), lambda i: (0, i)),
        out_shape=jax.ShapeDtypeStruct((D, N), jnp.float32),
    )(toka, tokb, flags, cons, fW1, fb1, fW2, fb2, cW1, cb1, cW2, cb2,
      ln_g, ln_b, oW1, ob1, oW2, ob2)


def kernel(global_context, piece_context, flags, consequence, square_emb,
           kind_emb, promo_emb, meta_emb, null_piece, fW1, fb1, fW2, fb2,
           cW1, cb1, cW2, cb2, ln_g, ln_b, oW1, ob1, oW2, ob2, moving_idx,
           target_idx, from_sq, to_sq, move_kind, promo_idx, meta_idx):
    piece_full = jnp.concatenate([piece_context, null_piece[None, :]], axis=0)
    pp = (piece_context[:, None, :] + piece_full[None, :, :]).reshape(-1, D)
    ss = (square_emb[:, None, :] + square_emb[None, :, :]).reshape(-1, D)
    kp = (kind_emb[:, None, :] + promo_emb[None, :, :]).reshape(-1, D)
    me = meta_emb + global_context
    i32 = jnp.int32
    raw = (moving_idx.astype(i32), target_idx.astype(i32),
           from_sq.astype(i32), to_sq.astype(i32),
           move_kind.astype(i32), promo_idx.astype(i32),
           meta_idx.astype(i32))

    toka, tokb = _sc_gather_sum((pp, ss, kp, me), raw)

    r1 = lambda v: v[None, :]
    out_t = _tc_encode(toka, tokb, flags.T, consequence.T,
                       fW1, r1(fb1), fW2.T, r1(fb2),
                       cW1, r1(cb1), cW2.T, r1(cb2),
                       r1(ln_g), r1(ln_b),
                       oW1, r1(ob1), oW2.T, ob2[:, None])
    return out_t.T


# speedup vs baseline: 3.3884x; 1.1259x over previous
import functools

import jax
import jax.numpy as jnp
from jax import lax
from jax.experimental import pallas as pl
from jax.experimental.pallas import tpu as pltpu
from jax.experimental.pallas import tpu_sc as plsc

N = 16384
D = 192
H = 384
P = 64
N_SQ = 65
N_KIND = 49
N_PROMO = 5
N_META = 129

NC = 2
NS = 16
NW = NC * NS
RPW = N // NW
C = 128
NCHUNK = RPW // C
NSTR = 4
VL = 16


def _gelu(x):
    return 0.5 * x * (1.0 + lax.erf(x * 0.7071067811865476))


def _sc_gather_sum(tables, raw_idx):
    mesh = plsc.VectorSubcoreMesh(core_axis_name="c", subcore_axis_name="s")

    @functools.partial(
        pl.kernel,
        mesh=mesh,
        out_type=(jax.ShapeDtypeStruct((N, 128), jnp.float32),
                  jax.ShapeDtypeStruct((N, 128), jnp.float32)),
        scratch_types=[
            pltpu.VMEM((6, RPW), jnp.int32),
            pltpu.VMEM((NSTR, RPW), jnp.int32),
            pltpu.VMEM((RPW, D), jnp.float32),
            pltpu.SemaphoreType.DMA,
        ],
        compiler_params=pltpu.CompilerParams(use_tc_tiling_on_sc=False),
    )
    def k(t0_hbm, t1_hbm, t2_hbm, t3_hbm, mov_hbm, tgt_hbm, frm_hbm,
          to_hbm, knd_hbm, prm_hbm, met_hbm, outa_hbm, outb_hbm, raw_v,
          idx_v, acc_v, sem):
        tabs = (t0_hbm, t1_hbm, t2_hbm, t3_hbm)
        wid = lax.axis_index("s") * NC + lax.axis_index("c")
        base = wid * RPW
        sl = pl.ds(base, RPW)
        incs = [
            pltpu.async_copy(r.at[sl], raw_v.at[j], sem)
            for j, r in enumerate(
                (mov_hbm, tgt_hbm, frm_hbm, to_hbm, knd_hbm, prm_hbm))
        ]
        incs.append(pltpu.async_copy(met_hbm.at[sl], idx_v.at[3], sem))
        for cp in incs:
            cp.wait()
        for i in range(RPW // VL):
            v = pl.ds(i * VL, VL)
            idx_v[0, v] = raw_v[0, v] * (P + 1) + raw_v[1, v]
            idx_v[1, v] = raw_v[2, v] * N_SQ + raw_v[3, v]
            idx_v[2, v] = raw_v[4, v] * N_PROMO + raw_v[5, v]

        fills = [
            pltpu.async_copy(
                tabs[0].at[idx_v.at[0, pl.ds(ci * C, C)]],
                acc_v.at[pl.ds(ci * C, C)], sem)
            for ci in range(NCHUNK)
        ]
        for cp in fills:
            cp.wait()
        adds = [
            pltpu.async_copy(
                tabs[t].at[idx_v.at[t, pl.ds(ci * C, C)]],
                acc_v.at[pl.ds(ci * C, C)], sem, add=True)
            for t in range(1, NSTR)
            for ci in range(NCHUNK)
        ]
        for cp in adds:
            cp.wait()
        pltpu.sync_copy(acc_v.at[:, pl.ds(0, 128)], outa_hbm.at[sl])
        pltpu.sync_copy(acc_v.at[:, pl.ds(128, 64)],
                        outb_hbm.at[sl, pl.ds(0, 64)])

    return k(*tables, *raw_idx)


def _tc_mlp_body(flags_ref, cons_ref, fW1_ref, fb1_ref, fW2_ref, fb2_ref,
                 cW1_ref, cb1_ref, cW2_ref, cb2_ref, out_ref):
    f32 = jnp.float32
    t_lhs = (((0,), (0,)), ((), ()))
    t_rhs = (((1,), (1,)), ((), ()))
    h1 = _gelu(lax.dot_general(flags_ref[...], fW1_ref[...], t_lhs,
                               preferred_element_type=f32) + fb1_ref[...])
    fc = lax.dot_general(h1, fW2_ref[...], t_rhs,
                         preferred_element_type=f32) + fb2_ref[...]
    h2 = _gelu(lax.dot_general(cons_ref[...], cW1_ref[...], t_lhs,
                               preferred_element_type=f32) + cb1_ref[...])
    out_ref[...] = fc + lax.dot_general(h2, cW2_ref[...], t_rhs,
                                        preferred_element_type=f32) + cb2_ref[...]


def _tc_body(toka_ref, tokb_ref, fc_ref, ln_g_ref, ln_b_ref, oW1_ref,
             ob1_ref, oW2_ref, ob2_ref, out_ref):
    f32 = jnp.float32
    tok = jnp.concatenate([toka_ref[...], tokb_ref[:, :64]], axis=1)
    tok = tok + fc_ref[...]
    mu = jnp.mean(tok, axis=-1, keepdims=True)
    cen = tok - mu
    var = jnp.mean(cen * cen, axis=-1, keepdims=True)
    h = cen * jax.lax.rsqrt(var + 1e-5) * ln_g_ref[...] + ln_b_ref[...]
    h3 = _gelu(jnp.dot(h, oW1_ref[...], preferred_element_type=f32) + ob1_ref[...])
    out_ref[...] = (lax.dot_general(oW2_ref[...], h3,
                                    (((1,), (1,)), ((), ())),
                                    preferred_element_type=f32)
                    + ob2_ref[...])


def _full(shape):
    return pl.BlockSpec(shape, lambda i: tuple(0 for _ in shape))


def _tc_feature_mlps(flags, cons, fW1, fb1, fW2, fb2, cW1, cb1, cW2, cb2,
                     block_n=4096):
    in_specs = [
        pl.BlockSpec((7, block_n), lambda i: (0, i)),
        pl.BlockSpec((12, block_n), lambda i: (0, i)),
        _full((7, H)), _full((1, H)), _full((D, H)), _full((1, D)),
        _full((12, H)), _full((1, H)), _full((D, H)), _full((1, D)),
    ]
    return pl.pallas_call(
        _tc_mlp_body,
        grid=(N // block_n,),
        in_specs=in_specs,
        out_specs=pl.BlockSpec((block_n, D), lambda i: (i, 0)),
        out_shape=jax.ShapeDtypeStruct((N, D), jnp.float32),
    )(flags, cons, fW1, fb1, fW2, fb2, cW1, cb1, cW2, cb2)


def _tc_encode(toka, tokb, fc, ln_g, ln_b, oW1, ob1, oW2, ob2,
               block_n=4096):
    half_spec = pl.BlockSpec((block_n, 128), lambda i: (i, 0))
    in_specs = [
        half_spec, half_spec,
        pl.BlockSpec((block_n, D), lambda i: (i, 0)),
        _full((1, D)), _full((1, D)),
        _full((D, H)), _full((1, H)), _full((D, H)), _full((D, 1)),
    ]
    return pl.pallas_call(
        _tc_body,
        grid=(N // block_n,),
        in_specs=in_specs,
        out_specs=pl.BlockSpec((D, block_n), lambda i: (0, i)),
        out_shape=jax.ShapeDtypeStruct((D, N), jnp.float32),
    )(toka, tokb, fc, ln_g, ln_b, oW1, ob1, oW2, ob2)


def kernel(global_context, piece_context, flags, consequence, square_emb,
           kind_emb, promo_emb, meta_emb, null_piece, fW1, fb1, fW2, fb2,
           cW1, cb1, cW2, cb2, ln_g, ln_b, oW1, ob1, oW2, ob2, moving_idx,
           target_idx, from_sq, to_sq, move_kind, promo_idx, meta_idx):
    piece_full = jnp.concatenate([piece_context, null_piece[None, :]], axis=0)
    pp = (piece_context[:, None, :] + piece_full[None, :, :]).reshape(-1, D)
    ss = (square_emb[:, None, :] + square_emb[None, :, :]).reshape(-1, D)
    kp = (kind_emb[:, None, :] + promo_emb[None, :, :]).reshape(-1, D)
    me = meta_emb + global_context
    i32 = jnp.int32
    raw = (moving_idx.astype(i32), target_idx.astype(i32),
           from_sq.astype(i32), to_sq.astype(i32),
           move_kind.astype(i32), promo_idx.astype(i32),
           meta_idx.astype(i32))

    toka, tokb = _sc_gather_sum((pp, ss, kp, me), raw)

    r1 = lambda v: v[None, :]
    fc = _tc_feature_mlps(flags.T, consequence.T,
                          fW1, r1(fb1), fW2.T, r1(fb2),
                          cW1, r1(cb1), cW2.T, r1(cb2))
    out_t = _tc_encode(toka, tokb, fc, r1(ln_g), r1(ln_b),
                       oW1, r1(ob1), oW2.T, ob2[:, None])
    return out_t.T
